# Initial kernel scaffold; baseline (speedup 1.0000x reference)
import jax
import jax.numpy as jnp
from jax.experimental import pallas as pl


def kernel(data_x, data_edge_index, data_edge_attr, t_w1, t_b1, ln1_g, ln1_b, t_w2, t_b2, ln2_g, ln2_b, rgcn_wrel, rgcn_wself, rgcn_b, c1_w, c1_b, c2_w, c2_b):
    raise NotImplementedError("write your pallas kernel here")


# speedup vs baseline: 9.1515x; 9.1515x over previous
import functools

import jax
import jax.numpy as jnp
from jax import lax
from jax.experimental import pallas as pl
from jax.experimental.pallas import tpu as pltpu
from jax.experimental.pallas import tpu_sc as plsc

N = 10000
E = 320000
IN_DIM = 128
HID = 100
OUT_DIM = 40

NPAD = 10240
NT = 32
ROWS_PER_TILE = NPAD // 16
CHUNK = 128
NCHUNK = 80
EPAD = NT * NCHUNK * CHUNK
F1 = 112
F3 = 48
BLK = 1024
GRID = NPAD // BLK


def _ln(h, g, b):
    mu = jnp.mean(h, axis=-1, keepdims=True)
    d = h - mu
    var = jnp.mean(d * d, axis=-1, keepdims=True)
    return d * lax.rsqrt(var + 1e-5) * g + b


def _k1_body(x_ref, w1_ref, b1_ref, g1_ref, lb1_ref, w2_ref, b2_ref, g2_ref,
             lb2_ref, wrel_ref, wself_ref, rb_ref, t1_ref, xs_ref):
    x = x_ref[...]
    h = jnp.dot(x, w1_ref[...], preferred_element_type=jnp.float32) + b1_ref[...]
    h = jnp.maximum(_ln(h, g1_ref[...], lb1_ref[...]), 0.0)
    h = jnp.dot(h, w2_ref[...], preferred_element_type=jnp.float32) + b2_ref[...]
    h = jnp.maximum(_ln(h, g2_ref[...], lb2_ref[...]), 0.0)
    t = jnp.dot(h, wrel_ref[...], preferred_element_type=jnp.float32)
    col = lax.broadcasted_iota(jnp.int32, (BLK, F1), 1)
    t1_ref[...] = t + (col == HID).astype(jnp.float32)
    xs_ref[...] = jnp.dot(h, wself_ref[...], preferred_element_type=jnp.float32) + rb_ref[...]


def _k1(x_pad, w1, b1, g1, lb1, w2, b2, g2, lb2, wrelp, wselfp, rbp):
    full = lambda shape: pl.BlockSpec(shape, lambda i: (0, 0))
    return pl.pallas_call(
        _k1_body,
        grid=(GRID,),
        in_specs=[
            pl.BlockSpec((BLK, IN_DIM), lambda i: (i, 0)),
            full((IN_DIM, HID)), full((1, HID)), full((1, HID)), full((1, HID)),
            full((HID, HID)), full((1, HID)), full((1, HID)), full((1, HID)),
            full((HID, F1)), full((HID, F1)), full((1, F1)),
        ],
        out_specs=[pl.BlockSpec((BLK, F1), lambda i: (i, 0)),
                   pl.BlockSpec((BLK, F1), lambda i: (i, 0))],
        out_shape=[jax.ShapeDtypeStruct((NPAD, F1), jnp.float32),
                   jax.ShapeDtypeStruct((NPAD, F1), jnp.float32)],
    )(x_pad, w1, b1, g1, lb1, w2, b2, g2, lb2, wrelp, wselfp, rbp)


def _k2_body(acc_ref, xs_ref, c1w_ref, t2_ref):
    s = acc_ref[0] + acc_ref[1]
    deg = s[:, HID:HID + 1]
    xr = jnp.maximum(xs_ref[...] + s * (1.0 / jnp.maximum(deg, 1.0)), 0.0)
    dinv = lax.rsqrt(deg + 1.0)
    t2 = jnp.dot(xr, c1w_ref[...], preferred_element_type=jnp.float32) * dinv
    col = lax.broadcasted_iota(jnp.int32, (BLK, F1), 1)
    t2_ref[...] = jnp.where(col == HID, dinv, t2)


def _k2(acc1, xselfb, c1wp):
    return pl.pallas_call(
        _k2_body,
        grid=(GRID,),
        in_specs=[
            pl.BlockSpec((2, BLK, F1), lambda i: (0, i, 0)),
            pl.BlockSpec((BLK, F1), lambda i: (i, 0)),
            pl.BlockSpec((F1, F1), lambda i: (0, 0)),
        ],
        out_specs=pl.BlockSpec((BLK, F1), lambda i: (i, 0)),
        out_shape=jax.ShapeDtypeStruct((NPAD, F1), jnp.float32),
    )(acc1, xselfb, c1wp)


def _k3_body(acc_ref, t2_ref, c1b_ref, c2w_ref, t3_ref):
    s = acc_ref[0] + acc_ref[1]
    t2 = t2_ref[...]
    dinv = t2[:, HID:HID + 1]
    xc = jnp.maximum(dinv * (s + t2) + c1b_ref[...], 0.0)
    t3 = jnp.dot(xc, c2w_ref[...], preferred_element_type=jnp.float32) * dinv
    col = lax.broadcasted_iota(jnp.int32, (BLK, F3), 1)
    t3_ref[...] = jnp.where(col == OUT_DIM, dinv, t3)


def _k3(acc2, table2, c1bp, c2wp):
    return pl.pallas_call(
        _k3_body,
        grid=(GRID,),
        in_specs=[
            pl.BlockSpec((2, BLK, F1), lambda i: (0, i, 0)),
            pl.BlockSpec((BLK, F1), lambda i: (i, 0)),
            pl.BlockSpec((1, F1), lambda i: (0, 0)),
            pl.BlockSpec((F1, F3), lambda i: (0, 0)),
        ],
        out_specs=pl.BlockSpec((BLK, F3), lambda i: (i, 0)),
        out_shape=jax.ShapeDtypeStruct((NPAD, F3), jnp.float32),
    )(acc2, table2, c1bp, c2wp)


def _k4_body(acc_ref, t3_ref, c2b_ref, out_ref):
    s = acc_ref[0] + acc_ref[1]
    t3 = t3_ref[...]
    dinv = t3[:, OUT_DIM:OUT_DIM + 1]
    logits = dinv * (s[:, :OUT_DIM] + t3[:, :OUT_DIM]) + c2b_ref[...]
    m = jnp.max(logits, axis=-1, keepdims=True)
    lse = jnp.log(jnp.sum(jnp.exp(logits - m), axis=-1, keepdims=True)) + m
    out_ref[...] = logits - lse


def _k4(acc3, table3, c2br):
    return pl.pallas_call(
        _k4_body,
        grid=(GRID,),
        in_specs=[
            pl.BlockSpec((2, BLK, F3), lambda i: (0, i, 0)),
            pl.BlockSpec((BLK, F3), lambda i: (i, 0)),
            pl.BlockSpec((1, OUT_DIM), lambda i: (0, 0)),
        ],
        out_specs=pl.BlockSpec((BLK, OUT_DIM), lambda i: (i, 0)),
        out_shape=jax.ShapeDtypeStruct((NPAD, OUT_DIM), jnp.float32),
    )(acc3, table3, c2br)


def _sc_pass(table, src3, dst3, zeros):
    f = table.shape[1]
    mesh = plsc.VectorSubcoreMesh(core_axis_name="c", subcore_axis_name="s",
                                  num_cores=2, num_subcores=16)

    @functools.partial(
        pl.kernel,
        out_type=jax.ShapeDtypeStruct((2, NPAD, f), jnp.float32),
        mesh=mesh,
        compiler_params=pltpu.CompilerParams(use_tc_tiling_on_sc=False),
        scratch_types=[
            pltpu.VMEM((CHUNK,), jnp.int32),
            pltpu.VMEM((CHUNK,), jnp.int32),
            pltpu.VMEM((CHUNK, f), jnp.float32),
            pltpu.VMEM_SHARED((NPAD, f), jnp.float32),
            pltpu.SemaphoreType.DMA,
        ],
    )
    def run(table_hbm, src_hbm, dst_hbm, zeros_hbm, out_hbm,
            src_c, dst_c, rows_v, acc_sh, sem):
        cid = lax.axis_index("c")
        sid = lax.axis_index("s")
        wid = cid * 16 + sid
        for k in range(16):
            @pl.when(sid == k)
            def _():
                pltpu.sync_copy(zeros_hbm,
                                acc_sh.at[pl.ds(k * ROWS_PER_TILE, ROWS_PER_TILE)])
        plsc.subcore_barrier()

        def chunk(j, carry):
            pltpu.sync_copy(src_hbm.at[wid, j], src_c)
            pltpu.sync_copy(dst_hbm.at[wid, j], dst_c)
            pltpu.async_copy(table_hbm.at[src_c], rows_v, sem).wait()
            pltpu.sync_copy(rows_v, acc_sh.at[dst_c], add=True)
            return carry

        lax.fori_loop(0, NCHUNK, chunk, 0)
        plsc.subcore_barrier()
        for k in range(16):
            @pl.when(sid == k)
            def _():
                pltpu.sync_copy(acc_sh.at[pl.ds(k * ROWS_PER_TILE, ROWS_PER_TILE)],
                                out_hbm.at[cid, pl.ds(k * ROWS_PER_TILE, ROWS_PER_TILE)])

    return run(table, src3, dst3, zeros)


def kernel(data_x, data_edge_index, data_edge_attr, t_w1, t_b1, ln1_g, ln1_b,
           t_w2, t_b2, ln2_g, ln2_b, rgcn_wrel, rgcn_wself, rgcn_b, c1_w,
           c1_b, c2_w, c2_b):
    f32 = jnp.float32
    x_pad = jnp.pad(data_x, ((0, NPAD - N), (0, 0)))
    row = lambda v: v.reshape(1, -1).astype(f32)
    wrelp = jnp.pad(rgcn_wrel[0], ((0, 0), (0, F1 - HID)))
    wselfp = jnp.pad(rgcn_wself, ((0, 0), (0, F1 - HID)))
    rbp = row(jnp.pad(rgcn_b, (0, F1 - HID)))
    c1wp = jnp.pad(c1_w, ((0, F1 - HID), (0, F1 - HID)))
    c1bp = row(jnp.pad(c1_b, (0, F1 - HID)))
    c2wp = jnp.pad(c2_w, ((0, F1 - HID), (0, F3 - OUT_DIM)))
    c2br = row(c2_b)

    pad = jnp.full((EPAD - E,), N, jnp.int32)
    src3 = jnp.concatenate([data_edge_index[0].astype(jnp.int32), pad]
                           ).reshape(NT, NCHUNK, CHUNK)
    dst3 = jnp.concatenate([data_edge_index[1].astype(jnp.int32), pad]
                           ).reshape(NT, NCHUNK, CHUNK)
    zeros1 = jnp.zeros((ROWS_PER_TILE, F1), f32)
    zeros3 = jnp.zeros((ROWS_PER_TILE, F3), f32)

    table1, xselfb = _k1(x_pad, t_w1, row(t_b1), row(ln1_g), row(ln1_b),
                         t_w2, row(t_b2), row(ln2_g), row(ln2_b),
                         wrelp, wselfp, rbp)
    acc1 = _sc_pass(table1, src3, dst3, zeros1)
    table2 = _k2(acc1, xselfb, c1wp)
    acc2 = _sc_pass(table2, src3, dst3, zeros1)
    table3 = _k3(acc2, table2, c1bp, c2wp)
    acc3 = _sc_pass(table3, src3, dst3, zeros3)
    out = _k4(acc3, table3, c2br)
    return out[:N]
